# baseline (device time: 121456 ns/iter reference)
import jax
import jax.numpy as jnp
from jax import lax
from jax.experimental import pallas as pl
from jax.experimental.pallas import tpu as pltpu

N_DEV = 4
SQ = 1024
SKV = 1024
H_PER = 8
DH = 128
D_MODEL = 1024
CHUNK = SQ // N_DEV
N_STEPS = 2 * (N_DEV - 1)
SCALE = 0.08838834764831843


def kernel(x, Wq, K_ext, V_ext, Wo):
    my = lax.axis_index("i")
    xb = x[0].astype(jnp.bfloat16)
    wq = Wq.astype(jnp.bfloat16)
    wo = Wo.astype(jnp.bfloat16)
    k = lax.dynamic_slice_in_dim(K_ext[0], my * H_PER, H_PER, axis=1)
    v = lax.dynamic_slice_in_dim(V_ext[0], my * H_PER, H_PER, axis=1)
    k = jnp.transpose(k, (1, 0, 2)).astype(jnp.bfloat16)
    v = jnp.transpose(v, (1, 0, 2)).astype(jnp.bfloat16)

    def body(x_ref, wq_ref, k_ref, v_ref, wo_ref, out_ref,
             comm_ref, send_sems, recv_sems):
        my_pos = lax.axis_index("i")
        left = lax.rem(my_pos + N_DEV - 1, N_DEV)
        right = lax.rem(my_pos + 1, N_DEV)

        barrier_sem = pltpu.get_barrier_semaphore()
        for nbr in (left, right):
            pl.semaphore_signal(
                barrier_sem, inc=1,
                device_id=(nbr,), device_id_type=pl.DeviceIdType.MESH,
            )
        pl.semaphore_wait(barrier_sem, 2)

        row_ids = lax.broadcasted_iota(jnp.int32, (SQ, SKV), 0)
        col_ids = lax.broadcasted_iota(jnp.int32, (SQ, SKV), 1)
        mask = ((row_ids // 64) % 4) == ((col_ids // 64) % 4)

        q = jnp.dot(x_ref[...], wq_ref[...], preferred_element_type=jnp.float32)
        q = (q * SCALE).astype(jnp.bfloat16)

        partial = jnp.zeros((SQ, D_MODEL), jnp.float32)
        for h in range(H_PER):
            qh = q[:, h * DH:(h + 1) * DH]
            s = lax.dot_general(
                qh, k_ref[h], (((1,), (1,)), ((), ())),
                preferred_element_type=jnp.float32,
            )
            s = jnp.where(mask, s, -1e9)
            m = jnp.max(s, axis=-1, keepdims=True)
            w = jnp.exp(s - m)
            w = w / jnp.sum(w, axis=-1, keepdims=True)
            ctx = jnp.dot(w.astype(jnp.bfloat16), v_ref[h],
                          preferred_element_type=jnp.float32)
            partial = partial + jnp.dot(
                ctx.astype(jnp.bfloat16), wo_ref[h * DH:(h + 1) * DH, :],
                preferred_element_type=jnp.float32,
            )
        out_ref[...] = partial

        for st in range(N_DEV - 1):
            send_c = lax.rem(my_pos - st + N_DEV, N_DEV)
            recv_c = lax.rem(my_pos - st - 1 + N_DEV, N_DEV)
            rdma = pltpu.make_async_remote_copy(
                src_ref=out_ref.at[pl.ds(send_c * CHUNK, CHUNK), :],
                dst_ref=comm_ref.at[st],
                send_sem=send_sems.at[st],
                recv_sem=recv_sems.at[st],
                device_id=(right,),
                device_id_type=pl.DeviceIdType.MESH,
            )
            rdma.start()
            rdma.wait()
            out_ref[pl.ds(recv_c * CHUNK, CHUNK), :] = (
                out_ref[pl.ds(recv_c * CHUNK, CHUNK), :] + comm_ref[st]
            )

        for st in range(N_DEV - 1):
            step = (N_DEV - 1) + st
            send_c = lax.rem(my_pos + 1 - st + N_DEV, N_DEV)
            rdma = pltpu.make_async_remote_copy(
                src_ref=out_ref.at[pl.ds(send_c * CHUNK, CHUNK), :],
                dst_ref=out_ref.at[pl.ds(send_c * CHUNK, CHUNK), :],
                send_sem=send_sems.at[step],
                recv_sem=recv_sems.at[step],
                device_id=(right,),
                device_id_type=pl.DeviceIdType.MESH,
            )
            rdma.start()
            rdma.wait()

    out = pl.pallas_call(
        body,
        out_shape=jax.ShapeDtypeStruct((SQ, D_MODEL), jnp.float32),
        in_specs=[pl.BlockSpec(memory_space=pltpu.VMEM)] * 5,
        out_specs=pl.BlockSpec(memory_space=pltpu.VMEM),
        scratch_shapes=[
            pltpu.VMEM((N_DEV - 1, CHUNK, D_MODEL), jnp.float32),
            pltpu.SemaphoreType.DMA((N_STEPS,)),
            pltpu.SemaphoreType.DMA((N_STEPS,)),
        ],
        compiler_params=pltpu.CompilerParams(collective_id=0),
    )(xb, wq, k, v, wo)
    return out[None]


# device time: 68857 ns/iter; 1.7639x vs baseline; 1.7639x over previous
import jax
import jax.numpy as jnp
from jax import lax
from jax.experimental import pallas as pl
from jax.experimental.pallas import tpu as pltpu

N_DEV = 4
SQ = 1024
H_PER = 8
DH = 128
D_MODEL = 1024
CHUNK = SQ // N_DEV
N_STEPS = 2 * (N_DEV - 1)
SCALE = 0.08838834764831843


def _perm_rows(a):
    s = a.shape
    return a.reshape(4, 4, 64, *s[1:]).swapaxes(0, 1).reshape(s)


def kernel(x, Wq, K_ext, V_ext, Wo):
    my = lax.axis_index("i")
    xp = _perm_rows(x[0]).astype(jnp.bfloat16)
    wq = Wq.astype(jnp.bfloat16)
    wo = Wo.astype(jnp.bfloat16)
    k = lax.dynamic_slice_in_dim(K_ext[0], my * H_PER, H_PER, axis=1)
    v = lax.dynamic_slice_in_dim(V_ext[0], my * H_PER, H_PER, axis=1)
    k = jnp.transpose(_perm_rows(k), (1, 0, 2)).astype(jnp.bfloat16)
    v = jnp.transpose(_perm_rows(v), (1, 0, 2)).astype(jnp.bfloat16)

    def body(x_ref, wq_ref, k_ref, v_ref, wo_ref, out_ref,
             send_ref, recv_ref, ag_ref, send_sems, recv_sems):
        my_pos = lax.axis_index("i")
        left = lax.rem(my_pos + N_DEV - 1, N_DEV)
        right = lax.rem(my_pos + 1, N_DEV)

        barrier_sem = pltpu.get_barrier_semaphore()
        for nbr in (left, right):
            pl.semaphore_signal(
                barrier_sem, inc=1,
                device_id=(nbr,), device_id_type=pl.DeviceIdType.MESH,
            )
        pl.semaphore_wait(barrier_sem, 2)

        def compute_chunk(c):
            coff = c * CHUNK
            qc = jnp.dot(x_ref[pl.ds(coff, CHUNK), :], wq_ref[...],
                         preferred_element_type=jnp.float32)
            qc = (qc * SCALE).astype(jnp.bfloat16)
            ctxs = []
            for h in range(H_PER):
                kh = k_ref[h, pl.ds(coff, CHUNK), :]
                vh = v_ref[h, pl.ds(coff, CHUNK), :]
                s = lax.dot_general(
                    qc[:, h * DH:(h + 1) * DH], kh,
                    (((1,), (1,)), ((), ())),
                    preferred_element_type=jnp.float32,
                )
                m = jnp.max(s, axis=-1, keepdims=True)
                w = jnp.exp(s - m)
                w = (w / jnp.sum(w, axis=-1, keepdims=True)).astype(jnp.bfloat16)
                ctxs.append(
                    jnp.dot(w, vh, preferred_element_type=jnp.float32)
                    .astype(jnp.bfloat16)
                )
            ctx = jnp.concatenate(ctxs, axis=1)
            return jnp.dot(ctx, wo_ref[...],
                           preferred_element_type=jnp.float32)

        def rs_rdma(st):
            return pltpu.make_async_remote_copy(
                src_ref=send_ref.at[st],
                dst_ref=recv_ref.at[st],
                send_sem=send_sems.at[st],
                recv_sem=recv_sems.at[st],
                device_id=(right,),
                device_id_type=pl.DeviceIdType.MESH,
            )

        acc = compute_chunk(my_pos)
        send_ref[0, :, :] = acc.astype(jnp.bfloat16)
        rdma = rs_rdma(0)
        rdma.start()
        for st in range(1, N_DEV - 1):
            c = lax.rem(my_pos - st + N_DEV, N_DEV)
            p = compute_chunk(c)
            rdma.wait()
            s = p + recv_ref[st - 1, :, :].astype(jnp.float32)
            send_ref[st, :, :] = s.astype(jnp.bfloat16)
            rdma = rs_rdma(st)
            rdma.start()
        owned_c = lax.rem(my_pos + 1, N_DEV)
        p = compute_chunk(owned_c)
        rdma.wait()
        owned = p + recv_ref[N_DEV - 2, :, :].astype(jnp.float32)
        out_ref[pl.ds(owned_c * CHUNK, CHUNK), :] = owned

        ag_ref[owned_c, :, :] = owned.astype(jnp.bfloat16)
        for st in range(N_DEV - 1):
            step = (N_DEV - 1) + st
            send_c = lax.rem(my_pos + 1 - st + N_DEV, N_DEV)
            rdma = pltpu.make_async_remote_copy(
                src_ref=ag_ref.at[send_c],
                dst_ref=ag_ref.at[send_c],
                send_sem=send_sems.at[step],
                recv_sem=recv_sems.at[step],
                device_id=(right,),
                device_id_type=pl.DeviceIdType.MESH,
            )
            rdma.start()
            if st > 0:
                prev_c = lax.rem(my_pos - st + 1 + N_DEV, N_DEV)
                out_ref[pl.ds(prev_c * CHUNK, CHUNK), :] = (
                    ag_ref[prev_c, :, :].astype(jnp.float32)
                )
            rdma.wait()
        last_c = lax.rem(my_pos - (N_DEV - 2) + N_DEV, N_DEV)
        out_ref[pl.ds(last_c * CHUNK, CHUNK), :] = (
            ag_ref[last_c, :, :].astype(jnp.float32)
        )

    out = pl.pallas_call(
        body,
        out_shape=jax.ShapeDtypeStruct((SQ, D_MODEL), jnp.float32),
        in_specs=[pl.BlockSpec(memory_space=pltpu.VMEM)] * 5,
        out_specs=pl.BlockSpec(memory_space=pltpu.VMEM),
        scratch_shapes=[
            pltpu.VMEM((N_DEV - 1, CHUNK, D_MODEL), jnp.bfloat16),
            pltpu.VMEM((N_DEV - 1, CHUNK, D_MODEL), jnp.bfloat16),
            pltpu.VMEM((N_DEV, CHUNK, D_MODEL), jnp.bfloat16),
            pltpu.SemaphoreType.DMA((N_STEPS,)),
            pltpu.SemaphoreType.DMA((N_STEPS,)),
        ],
        compiler_params=pltpu.CompilerParams(collective_id=0),
    )(xp, wq, k, v, wo)
    return _perm_rows(out)[None]
